# baseline (speedup 1.0000x reference)
import jax
import jax.numpy as jnp
from jax.experimental import pallas as pl
from jax.experimental.pallas import tpu as pltpu
from jax.experimental.pallas import tpu_sc as plsc

B = 16
T = 512
H = 128
FWD = 48
RETRO = 16
NC = T - 3
NEG = float("-inf")

_GATHER_WINDOW = 128


def _sc_gather(emb, seq_flat):
    n = seq_flat.shape[1]
    mesh = plsc.VectorSubcoreMesh(core_axis_name="core",
                                  subcore_axis_name="subcore")

    @pl.kernel(out_type=jax.ShapeDtypeStruct((n, emb.shape[1]), emb.dtype),
               mesh=mesh)
    def gather_kernel(x_hbm, i_hbm, o_hbm):
        def body(i_vmem, o_vmem):
            pltpu.sync_copy(x_hbm.at[i_vmem.at[0]], o_vmem)

        pltpu.emit_pipeline(
            body,
            grid=(n // _GATHER_WINDOW,),
            in_specs=[pl.BlockSpec((1, _GATHER_WINDOW),
                                   index_map=lambda i: (0, i))],
            out_specs=[pl.BlockSpec((_GATHER_WINDOW, emb.shape[1]),
                                    index_map=lambda i: (i, 0))],
            core_axis_name=("core", "subcore"),
            dimension_semantics=(pltpu.PARALLEL,),
        )(i_hbm, o_hbm)

    return gather_kernel(emb, seq_flat)


def _middle_body(h0_ref, ffw1_ref, ffb1_ref, ffw2_ref, ffb2_ref,
                 lng_ref, lnb_ref, fgw_ref, fgb_ref, w1a_ref, w1b_ref,
                 rb1_ref, rw2_ref, qw_ref, qb_ref, ctx_ref):
    h0 = h0_ref[...]
    ff1 = jnp.maximum(
        jnp.dot(h0, ffw1_ref[...], preferred_element_type=jnp.float32)
        + ffb1_ref[...], 0.0)
    ff = jnp.dot(ff1, ffw2_ref[...],
                 preferred_element_type=jnp.float32) + ffb2_ref[...]
    x = h0 + ff
    mu = jnp.mean(x, axis=-1, keepdims=True)
    xc = x - mu
    var = jnp.mean(xc * xc, axis=-1, keepdims=True)
    hidden = xc / jnp.sqrt(var + 1e-5) * lng_ref[...] + lnb_ref[...]

    h3 = hidden.reshape(B, T, H)
    iota = jax.lax.broadcasted_iota(jnp.int32, (B, T), 1)
    validc = iota < NC

    fwd_s = jnp.dot(hidden, fgw_ref[...],
                    preferred_element_type=jnp.float32).reshape(B, T)
    fwd_s = fwd_s + fgb_ref[0, 0]
    fwd_s = jnp.where(validc, fwd_s, NEG)

    def extract(scores, k):
        def body(_, carry):
            sc, m = carry
            mx = jnp.max(sc, axis=1, keepdims=True)
            eq = sc == mx
            idx = jnp.min(jnp.where(eq, iota, T), axis=1, keepdims=True)
            sel = iota == idx
            return jnp.where(sel, NEG, sc), jnp.maximum(
                m, jnp.where(sel, 1.0, 0.0))
        _, mask = jax.lax.fori_loop(
            0, k, body, (scores, jnp.zeros((B, T), jnp.float32)))
        return mask > 0.5

    fwd_mask = extract(fwd_s, FWD)

    context = jnp.mean(h3, axis=1)
    g1lin = jnp.dot(hidden, w1a_ref[...],
                    preferred_element_type=jnp.float32).reshape(B, T, H)
    cb = jnp.dot(context, w1b_ref[...],
                 preferred_element_type=jnp.float32) + rb1_ref[...]
    g1 = jnp.maximum(g1lin + cb.reshape(B, 1, H), 0.0)
    z = jnp.dot(g1.reshape(B * T, H), rw2_ref[...],
                preferred_element_type=jnp.float32).reshape(B, T)
    z = jnp.where(
        jnp.logical_and(validc, jnp.logical_not(fwd_mask)), z, NEG)
    retro_mask = extract(z, RETRO)

    sel = jnp.logical_or(fwd_mask, retro_mask)

    q = jnp.dot(h3[:, T - 2, :], qw_ref[...],
                preferred_element_type=jnp.float32) + qb_ref[...]
    h3r = h3.astype(jnp.bfloat16).astype(jnp.float32)
    qr = q.astype(jnp.bfloat16).astype(jnp.float32)
    att = jnp.sum(h3r * qr.reshape(B, 1, H), axis=-1)
    att = jnp.where(sel, att, NEG)
    mx = jnp.max(att, axis=1, keepdims=True)
    e = jnp.exp(att - mx)
    attn = e / jnp.sum(e, axis=1, keepdims=True)
    ctx_ref[...] = jnp.sum(h3 * attn.reshape(B, T, 1), axis=1)


def _middle(h0, ffw1, ffb1, ffw2, ffb2, lng, lnb, fgw, fgb, w1a, w1b, rb1,
            rw2, qw, qb):
    return pl.pallas_call(
        _middle_body,
        out_shape=jax.ShapeDtypeStruct((B, H), jnp.float32),
    )(h0, ffw1, ffb1, ffw2, ffb2, lng, lnb, fgw, fgb, w1a, w1b, rb1,
      rw2, qw, qb)


_VTILE = 2048


def _vocab_body(ctx_ref, w_ref, b_ref, o_ref):
    o_ref[...] = jnp.dot(ctx_ref[...], w_ref[...],
                         preferred_element_type=jnp.float32) + b_ref[...]


def _vocab(ctx, out_w, out_b2):
    vocab = out_w.shape[1]
    return pl.pallas_call(
        _vocab_body,
        grid=(pl.cdiv(vocab, _VTILE),),
        in_specs=[
            pl.BlockSpec((B, H), lambda i: (0, 0)),
            pl.BlockSpec((H, _VTILE), lambda i: (0, i)),
            pl.BlockSpec((1, _VTILE), lambda i: (0, i)),
        ],
        out_specs=pl.BlockSpec((B, _VTILE), lambda i: (0, i)),
        out_shape=jax.ShapeDtypeStruct((B, vocab), jnp.float32),
        compiler_params=pltpu.CompilerParams(
            dimension_semantics=("arbitrary",)),
    )(ctx, out_w, out_b2)


def kernel(seq, emb, ff_w1, ff_b1, ff_w2, ff_b2, ln_g, ln_b, fg_w, fg_b,
           rev_w1, rev_b1, rev_w2, rev_b2, q_w, q_b, out_w, out_b):
    h0 = jnp.take(emb, seq.reshape(B * T), axis=0)
    ctx = _middle(
        h0,
        ff_w1, ff_b1.reshape(1, 2 * H), ff_w2, ff_b2.reshape(1, H),
        ln_g.reshape(1, H), ln_b.reshape(1, H),
        fg_w, fg_b.reshape(1, 1),
        rev_w1[:H], rev_w1[H:], rev_b1.reshape(1, H),
        rev_w2,
        q_w, q_b.reshape(1, H),
    )
    return _vocab(ctx, out_w, out_b.reshape(1, out_w.shape[1]))


# speedup vs baseline: 1.0511x; 1.0511x over previous
import jax
import jax.numpy as jnp
from jax.experimental import pallas as pl
from jax.experimental.pallas import tpu as pltpu
from jax.experimental.pallas import tpu_sc as plsc

B = 16
T = 512
H = 128
FWD = 48
RETRO = 16
NC = T - 3
NEG = float("-inf")

_GATHER_WINDOW = 128


def _sc_gather(emb, seq_flat):
    n = seq_flat.shape[1]
    mesh = plsc.VectorSubcoreMesh(core_axis_name="core",
                                  subcore_axis_name="subcore")

    @pl.kernel(out_type=jax.ShapeDtypeStruct((n, emb.shape[1]), emb.dtype),
               mesh=mesh)
    def gather_kernel(x_hbm, i_hbm, o_hbm):
        def body(i_vmem, o_vmem):
            pltpu.sync_copy(x_hbm.at[i_vmem.at[0]], o_vmem)

        pltpu.emit_pipeline(
            body,
            grid=(n // _GATHER_WINDOW,),
            in_specs=[pl.BlockSpec((1, _GATHER_WINDOW),
                                   index_map=lambda i: (0, i))],
            out_specs=[pl.BlockSpec((_GATHER_WINDOW, emb.shape[1]),
                                    index_map=lambda i: (i, 0))],
            core_axis_name=("core", "subcore"),
            dimension_semantics=(pltpu.PARALLEL,),
        )(i_hbm, o_hbm)

    return gather_kernel(emb, seq_flat)


def _middle_body(h0_ref, ffw1_ref, ffb1_ref, ffw2_ref, ffb2_ref,
                 lng_ref, lnb_ref, fgw_ref, fgb_ref, w1a_ref, w1b_ref,
                 rb1_ref, rw2_ref, qw_ref, qb_ref, ctx_ref):
    h0 = h0_ref[...]
    ff1 = jnp.maximum(
        jnp.dot(h0, ffw1_ref[...], preferred_element_type=jnp.float32)
        + ffb1_ref[...], 0.0)
    ff = jnp.dot(ff1, ffw2_ref[...],
                 preferred_element_type=jnp.float32) + ffb2_ref[...]
    x = h0 + ff
    mu = jnp.mean(x, axis=-1, keepdims=True)
    xc = x - mu
    var = jnp.mean(xc * xc, axis=-1, keepdims=True)
    hidden = xc / jnp.sqrt(var + 1e-5) * lng_ref[...] + lnb_ref[...]

    h3 = hidden.reshape(B, T, H)
    iota = jax.lax.broadcasted_iota(jnp.int32, (B, T), 1)
    validc = iota < NC

    def col_scores(mat, wcol_ref):
        wrep = jnp.broadcast_to(wcol_ref[...], (H, H))
        s = jnp.dot(mat, wrep, preferred_element_type=jnp.float32)
        return jnp.max(s.reshape(B, T, H), axis=-1)

    fwd_s = col_scores(hidden, fgw_ref)
    fwd_s = fwd_s + fgb_ref[0, 0]
    fwd_s = jnp.where(validc, fwd_s, NEG)

    def extract(scores, k):
        def body(_, carry):
            sc, m = carry
            mx = jnp.max(sc, axis=1, keepdims=True)
            eq = sc == mx
            idx = jnp.min(jnp.where(eq, iota, T), axis=1, keepdims=True)
            sel = iota == idx
            return jnp.where(sel, NEG, sc), jnp.maximum(
                m, jnp.where(sel, 1.0, 0.0))
        _, mask = jax.lax.fori_loop(
            0, k, body, (scores, jnp.zeros((B, T), jnp.float32)))
        return mask > 0.5

    fwd_mask = extract(fwd_s, FWD)

    context = jnp.mean(h3, axis=1)
    g1lin = jnp.dot(hidden, w1a_ref[...],
                    preferred_element_type=jnp.float32).reshape(B, T, H)
    cb = jnp.dot(context, w1b_ref[...],
                 preferred_element_type=jnp.float32) + rb1_ref[...]
    g1 = jnp.maximum(g1lin + cb.reshape(B, 1, H), 0.0)
    z = col_scores(g1.reshape(B * T, H), rw2_ref)
    z = jnp.where(
        jnp.logical_and(validc, jnp.logical_not(fwd_mask)), z, NEG)
    retro_mask = extract(z, RETRO)

    sel = jnp.logical_or(fwd_mask, retro_mask)

    q = jnp.dot(h3[:, T - 2, :], qw_ref[...],
                preferred_element_type=jnp.float32) + qb_ref[...]
    h3r = h3.astype(jnp.bfloat16).astype(jnp.float32)
    qr = q.astype(jnp.bfloat16).astype(jnp.float32)
    att = jnp.sum(h3r * qr.reshape(B, 1, H), axis=-1)
    att = jnp.where(sel, att, NEG)
    mx = jnp.max(att, axis=1, keepdims=True)
    e = jnp.exp(att - mx)
    attn = e / jnp.sum(e, axis=1, keepdims=True)
    ctx_ref[...] = jnp.sum(h3 * attn.reshape(B, T, 1), axis=1)


def _middle(h0, ffw1, ffb1, ffw2, ffb2, lng, lnb, fgw, fgb, w1a, w1b, rb1,
            rw2, qw, qb):
    return pl.pallas_call(
        _middle_body,
        out_shape=jax.ShapeDtypeStruct((B, H), jnp.float32),
    )(h0, ffw1, ffb1, ffw2, ffb2, lng, lnb, fgw, fgb, w1a, w1b, rb1,
      rw2, qw, qb)


_VTILE = 2048


def _vocab_body(ctx_ref, w_ref, b_ref, o_ref):
    o_ref[...] = jnp.dot(ctx_ref[...], w_ref[...],
                         preferred_element_type=jnp.float32) + b_ref[...]


def _vocab(ctx, out_w_bf16, out_b2):
    vocab = out_w_bf16.shape[1]
    return pl.pallas_call(
        _vocab_body,
        grid=(pl.cdiv(vocab, _VTILE),),
        in_specs=[
            pl.BlockSpec((B, H), lambda i: (0, 0)),
            pl.BlockSpec((H, _VTILE), lambda i: (0, i)),
            pl.BlockSpec((1, _VTILE), lambda i: (0, i)),
        ],
        out_specs=pl.BlockSpec((B, _VTILE), lambda i: (0, i)),
        out_shape=jax.ShapeDtypeStruct((B, vocab), jnp.float32),
        compiler_params=pltpu.CompilerParams(
            dimension_semantics=("arbitrary",)),
    )(ctx, out_w_bf16, out_b2)


def kernel(seq, emb, ff_w1, ff_b1, ff_w2, ff_b2, ln_g, ln_b, fg_w, fg_b,
           rev_w1, rev_b1, rev_w2, rev_b2, q_w, q_b, out_w, out_b):
    h0 = _sc_gather(emb, seq.reshape(1, B * T).astype(jnp.int32))
    ctx = _middle(
        h0,
        ff_w1, ff_b1.reshape(1, 2 * H), ff_w2, ff_b2.reshape(1, H),
        ln_g.reshape(1, H), ln_b.reshape(1, H),
        fg_w, fg_b.reshape(1, 1),
        rev_w1[:H], rev_w1[H:], rev_b1.reshape(1, H),
        rev_w2,
        q_w, q_b.reshape(1, H),
    )
    return _vocab(ctx, out_w.astype(jnp.bfloat16),
                  out_b.reshape(1, out_w.shape[1]))


# speedup vs baseline: 1.2390x; 1.1788x over previous
import jax
import jax.numpy as jnp
from jax.experimental import pallas as pl
from jax.experimental.pallas import tpu as pltpu
from jax.experimental.pallas import tpu_sc as plsc

B = 16
T = 512
H = 128
FWD = 48
RETRO = 16
NC = T - 3
NEG = float("-inf")

_GATHER_WINDOW = 128


def _sc_gather(emb, seq_flat):
    n = seq_flat.shape[1]
    mesh = plsc.VectorSubcoreMesh(core_axis_name="core",
                                  subcore_axis_name="subcore")

    @pl.kernel(out_type=jax.ShapeDtypeStruct((n, emb.shape[1]), emb.dtype),
               mesh=mesh)
    def gather_kernel(x_hbm, i_hbm, o_hbm):
        def body(i_vmem, o_vmem):
            pltpu.sync_copy(x_hbm.at[i_vmem.at[0]], o_vmem)

        pltpu.emit_pipeline(
            body,
            grid=(n // _GATHER_WINDOW,),
            in_specs=[pl.BlockSpec((1, _GATHER_WINDOW),
                                   index_map=lambda i: (0, i))],
            out_specs=[pl.BlockSpec((_GATHER_WINDOW, emb.shape[1]),
                                    index_map=lambda i: (i, 0))],
            core_axis_name=("core", "subcore"),
            dimension_semantics=(pltpu.PARALLEL,),
        )(i_hbm, o_hbm)

    return gather_kernel(emb, seq_flat)


def _middle_body(h0_ref, ffw1_ref, ffb1_ref, ffw2_ref, ffb2_ref,
                 lng_ref, lnb_ref, fgw_ref, w1a_ref, w1b_ref,
                 rb1_ref, rw2_ref, qw_ref, qb_ref, ctx_ref):
    h0 = h0_ref[...]
    ff1 = jnp.maximum(
        jnp.dot(h0, ffw1_ref[...], preferred_element_type=jnp.float32)
        + ffb1_ref[...], 0.0)
    ff = jnp.dot(ff1, ffw2_ref[...],
                 preferred_element_type=jnp.float32) + ffb2_ref[...]
    x = h0 + ff
    mu = jnp.mean(x, axis=-1, keepdims=True)
    xc = x - mu
    var = jnp.mean(xc * xc, axis=-1, keepdims=True)
    hidden = xc / jnp.sqrt(var + 1e-5) * lng_ref[...] + lnb_ref[...]

    h3 = hidden.reshape(B, T, H)
    iota = jax.lax.broadcasted_iota(jnp.int32, (B, T), 1)
    validc = iota < NC

    def col_scores(mat, wcol_ref):
        wrep = jnp.broadcast_to(wcol_ref[...], (H, H))
        s = jnp.dot(mat, wrep, preferred_element_type=jnp.float32)
        return jnp.max(s.reshape(B, T, H), axis=-1)

    fwd_s = col_scores(hidden, fgw_ref)
    fwd_s = jnp.where(validc, fwd_s, NEG)

    def extract(scores, k):
        def body(_, carry):
            sc, m = carry
            mx = jnp.max(sc, axis=1, keepdims=True)
            eq = sc == mx
            idx = jnp.min(jnp.where(eq, iota, T), axis=1, keepdims=True)
            sel = iota == idx
            return jnp.where(sel, NEG, sc), jnp.maximum(
                m, jnp.where(sel, 1.0, 0.0))
        _, mask = jax.lax.fori_loop(
            0, k, body, (scores, jnp.zeros((B, T), jnp.float32)))
        return mask > 0.5

    fwd_mask = extract(fwd_s, FWD)

    context = jnp.mean(h3, axis=1)
    g1lin = jnp.dot(hidden, w1a_ref[...],
                    preferred_element_type=jnp.float32).reshape(B, T, H)
    cb = jnp.dot(context, w1b_ref[...],
                 preferred_element_type=jnp.float32) + rb1_ref[...]
    g1 = jnp.maximum(g1lin + cb.reshape(B, 1, H), 0.0)
    z = col_scores(g1.reshape(B * T, H), rw2_ref)
    z = jnp.where(
        jnp.logical_and(validc, jnp.logical_not(fwd_mask)), z, NEG)
    retro_mask = extract(z, RETRO)

    sel = jnp.logical_or(fwd_mask, retro_mask)

    q = jnp.dot(h3[:, T - 2, :], qw_ref[...],
                preferred_element_type=jnp.float32) + qb_ref[...]
    h3r = h3.astype(jnp.bfloat16).astype(jnp.float32)
    qr = q.astype(jnp.bfloat16).astype(jnp.float32)
    att = jnp.sum(h3r * qr.reshape(B, 1, H), axis=-1)
    att = jnp.where(sel, att, NEG)
    mx = jnp.max(att, axis=1, keepdims=True)
    e = jnp.exp(att - mx)
    attn = e / jnp.sum(e, axis=1, keepdims=True)
    ctx_ref[...] = jnp.sum(h3 * attn.reshape(B, T, 1), axis=1)


def _middle(h0, ffw1, ffb1, ffw2, ffb2, lng, lnb, fgw, w1a, w1b, rb1,
            rw2, qw, qb):
    return pl.pallas_call(
        _middle_body,
        out_shape=jax.ShapeDtypeStruct((B, H), jnp.float32),
    )(h0, ffw1, ffb1, ffw2, ffb2, lng, lnb, fgw, w1a, w1b, rb1,
      rw2, qw, qb)


_VTILE = 4096


def _vocab_body(ctx_ref, w_ref, b_ref, o_ref):
    o_ref[...] = jnp.dot(ctx_ref[...], w_ref[...],
                         preferred_element_type=jnp.float32) + b_ref[...]


def _vocab(ctx, out_w, out_b2):
    vocab = out_w.shape[1]
    return pl.pallas_call(
        _vocab_body,
        grid=(pl.cdiv(vocab, _VTILE),),
        in_specs=[
            pl.BlockSpec((B, H), lambda i: (0, 0)),
            pl.BlockSpec((H, _VTILE), lambda i: (0, i)),
            pl.BlockSpec((1, _VTILE), lambda i: (0, i)),
        ],
        out_specs=pl.BlockSpec((B, _VTILE), lambda i: (0, i)),
        out_shape=jax.ShapeDtypeStruct((B, vocab), jnp.float32),
        compiler_params=pltpu.CompilerParams(
            dimension_semantics=("arbitrary",)),
    )(ctx, out_w, out_b2)


def kernel(seq, emb, ff_w1, ff_b1, ff_w2, ff_b2, ln_g, ln_b, fg_w, fg_b,
           rev_w1, rev_b1, rev_w2, rev_b2, q_w, q_b, out_w, out_b):
    h0 = _sc_gather(emb, seq.reshape(1, B * T).astype(jnp.int32))
    ctx = _middle(
        h0,
        ff_w1, ff_b1.reshape(1, 2 * H), ff_w2, ff_b2.reshape(1, H),
        ln_g.reshape(1, H), ln_b.reshape(1, H),
        fg_w,
        rev_w1[:H], rev_w1[H:], rev_b1.reshape(1, H),
        rev_w2,
        q_w, q_b.reshape(1, H),
    )
    return _vocab(ctx, out_w, out_b.reshape(1, out_w.shape[1]))


# speedup vs baseline: 1.2456x; 1.0053x over previous
import jax
import jax.numpy as jnp
from jax.experimental import pallas as pl
from jax.experimental.pallas import tpu as pltpu
from jax.experimental.pallas import tpu_sc as plsc

B = 16
T = 512
H = 128
FWD = 48
RETRO = 16
NC = T - 3
NEG = float("-inf")

_GATHER_WINDOW = 256


def _sc_gather(emb, seq_flat):
    n = seq_flat.shape[1]
    mesh = plsc.VectorSubcoreMesh(core_axis_name="core",
                                  subcore_axis_name="subcore")

    @pl.kernel(out_type=jax.ShapeDtypeStruct((n, emb.shape[1]), emb.dtype),
               mesh=mesh)
    def gather_kernel(x_hbm, i_hbm, o_hbm):
        def body(i_vmem, o_vmem):
            pltpu.sync_copy(x_hbm.at[i_vmem.at[0]], o_vmem)

        pltpu.emit_pipeline(
            body,
            grid=(n // _GATHER_WINDOW,),
            in_specs=[pl.BlockSpec((1, _GATHER_WINDOW),
                                   index_map=lambda i: (0, i))],
            out_specs=[pl.BlockSpec((_GATHER_WINDOW, emb.shape[1]),
                                    index_map=lambda i: (i, 0))],
            core_axis_name=("core", "subcore"),
            dimension_semantics=(pltpu.PARALLEL,),
        )(i_hbm, o_hbm)

    return gather_kernel(emb, seq_flat)


def _middle_body(h0_ref, ffw1_ref, ffb1_ref, ffw2_ref, ffb2_ref,
                 lng_ref, lnb_ref, fgw_ref, rw1_ref,
                 rb1_ref, rw2_ref, qw_ref, qb_ref, ctx_ref):
    h0 = h0_ref[...]
    ff1 = jnp.maximum(
        jnp.dot(h0, ffw1_ref[...], preferred_element_type=jnp.float32)
        + ffb1_ref[...], 0.0)
    ff = jnp.dot(ff1, ffw2_ref[...],
                 preferred_element_type=jnp.float32) + ffb2_ref[...]
    x = h0 + ff
    mu = jnp.mean(x, axis=-1, keepdims=True)
    xc = x - mu
    var = jnp.mean(xc * xc, axis=-1, keepdims=True)
    hidden = xc / jnp.sqrt(var + 1e-5) * lng_ref[...] + lnb_ref[...]

    h3 = hidden.reshape(B, T, H)
    iota = jax.lax.broadcasted_iota(jnp.int32, (B, T), 1)
    validc = iota < NC

    def col_scores(mat, wcol_ref):
        wrep = jnp.broadcast_to(wcol_ref[...], (H, H))
        s = jnp.dot(mat, wrep, preferred_element_type=jnp.float32)
        return jnp.max(s.reshape(B, T, H), axis=-1)

    fwd_s = col_scores(hidden, fgw_ref)
    fwd_s = jnp.where(validc, fwd_s, NEG)

    def extract(scores, k):
        def body(_, carry):
            sc, m = carry
            mx = jnp.max(sc, axis=1, keepdims=True)
            eq = sc == mx
            idx = jnp.min(jnp.where(eq, iota, T), axis=1, keepdims=True)
            sel = iota == idx
            return jnp.where(sel, NEG, sc), jnp.maximum(
                m, jnp.where(sel, 1.0, 0.0))
        _, mask = jax.lax.fori_loop(
            0, k, body, (scores, jnp.zeros((B, T), jnp.float32)))
        return mask > 0.5

    fwd_mask = extract(fwd_s, FWD)

    context = jnp.mean(h3, axis=1)
    g1lin = jnp.dot(hidden, rw1_ref[0:H, :],
                    preferred_element_type=jnp.float32).reshape(B, T, H)
    cb = jnp.dot(context, rw1_ref[H:2 * H, :],
                 preferred_element_type=jnp.float32) + rb1_ref[...]
    g1 = jnp.maximum(g1lin + cb.reshape(B, 1, H), 0.0)
    z = col_scores(g1.reshape(B * T, H), rw2_ref)
    z = jnp.where(
        jnp.logical_and(validc, jnp.logical_not(fwd_mask)), z, NEG)
    retro_mask = extract(z, RETRO)

    sel = jnp.logical_or(fwd_mask, retro_mask)

    q = jnp.dot(h3[:, T - 2, :], qw_ref[...],
                preferred_element_type=jnp.float32) + qb_ref[...]
    h3r = h3.astype(jnp.bfloat16).astype(jnp.float32)
    qr = q.astype(jnp.bfloat16).astype(jnp.float32)
    att = jnp.sum(h3r * qr.reshape(B, 1, H), axis=-1)
    att = jnp.where(sel, att, NEG)
    mx = jnp.max(att, axis=1, keepdims=True)
    e = jnp.exp(att - mx)
    attn = e / jnp.sum(e, axis=1, keepdims=True)
    ctx_ref[...] = jnp.sum(h3 * attn.reshape(B, T, 1), axis=1)


def _middle(h0, ffw1, ffb1, ffw2, ffb2, lng, lnb, fgw, rw1, rb1,
            rw2, qw, qb):
    return pl.pallas_call(
        _middle_body,
        out_shape=jax.ShapeDtypeStruct((B, H), jnp.float32),
    )(h0, ffw1, ffb1, ffw2, ffb2, lng, lnb, fgw, rw1, rb1,
      rw2, qw, qb)


_VTILE = 4096


def _vocab_body(ctx_ref, w_ref, b_ref, o_ref):
    o_ref[...] = jnp.dot(ctx_ref[...], w_ref[...],
                         preferred_element_type=jnp.float32) + b_ref[...]


def _vocab(ctx, out_w, out_b2):
    vocab = out_w.shape[1]
    return pl.pallas_call(
        _vocab_body,
        grid=(pl.cdiv(vocab, _VTILE),),
        in_specs=[
            pl.BlockSpec((B, H), lambda i: (0, 0)),
            pl.BlockSpec((H, _VTILE), lambda i: (0, i)),
            pl.BlockSpec((1, _VTILE), lambda i: (0, i)),
        ],
        out_specs=pl.BlockSpec((B, _VTILE), lambda i: (0, i)),
        out_shape=jax.ShapeDtypeStruct((B, vocab), jnp.float32),
        compiler_params=pltpu.CompilerParams(
            dimension_semantics=("arbitrary",)),
    )(ctx, out_w, out_b2)


def kernel(seq, emb, ff_w1, ff_b1, ff_w2, ff_b2, ln_g, ln_b, fg_w, fg_b,
           rev_w1, rev_b1, rev_w2, rev_b2, q_w, q_b, out_w, out_b):
    h0 = _sc_gather(emb, seq.reshape(1, B * T).astype(jnp.int32))
    ctx = _middle(
        h0,
        ff_w1, ff_b1.reshape(1, 2 * H), ff_w2, ff_b2.reshape(1, H),
        ln_g.reshape(1, H), ln_b.reshape(1, H),
        fg_w,
        rev_w1, rev_b1.reshape(1, H),
        rev_w2,
        q_w, q_b.reshape(1, H),
    )
    return _vocab(ctx, out_w, out_b.reshape(1, out_w.shape[1]))


# speedup vs baseline: 1.2644x; 1.0151x over previous
import jax
import jax.numpy as jnp
from jax.experimental import pallas as pl
from jax.experimental.pallas import tpu as pltpu
from jax.experimental.pallas import tpu_sc as plsc

B = 16
T = 512
H = 128
FWD = 48
RETRO = 16
NC = T - 3
NEG = float("-inf")

_GATHER_WINDOW = 256


def _sc_gather(emb, seq_flat):
    n = seq_flat.shape[1]
    mesh = plsc.VectorSubcoreMesh(core_axis_name="core",
                                  subcore_axis_name="subcore")

    @pl.kernel(out_type=jax.ShapeDtypeStruct((n, emb.shape[1]), emb.dtype),
               mesh=mesh)
    def gather_kernel(x_hbm, i_hbm, o_hbm):
        def body(i_vmem, o_vmem):
            pltpu.sync_copy(x_hbm.at[i_vmem.at[0]], o_vmem)

        pltpu.emit_pipeline(
            body,
            grid=(n // _GATHER_WINDOW,),
            in_specs=[pl.BlockSpec((1, _GATHER_WINDOW),
                                   index_map=lambda i: (0, i))],
            out_specs=[pl.BlockSpec((_GATHER_WINDOW, emb.shape[1]),
                                    index_map=lambda i: (i, 0))],
            core_axis_name=("core", "subcore"),
            dimension_semantics=(pltpu.PARALLEL,),
        )(i_hbm, o_hbm)

    return gather_kernel(emb, seq_flat)


_VTILE = 4096


def _fused_body(h0_ref, ffw1_ref, ffb1_ref, ffw2_ref, ffb2_ref,
                lng_ref, lnb_ref, fgw_ref, rw1_ref,
                rb1_ref, rw2_ref, qw_ref, qb_ref,
                w_ref, b_ref, o_ref, ctx_ref):
    @pl.when(pl.program_id(0) == 0)
    def _compute_ctx():
        h0 = h0_ref[...]
        ff1 = jnp.maximum(
            jnp.dot(h0, ffw1_ref[...], preferred_element_type=jnp.float32)
            + ffb1_ref[...], 0.0)
        ff = jnp.dot(ff1, ffw2_ref[...],
                     preferred_element_type=jnp.float32) + ffb2_ref[...]
        x = h0 + ff
        mu = jnp.mean(x, axis=-1, keepdims=True)
        xc = x - mu
        var = jnp.mean(xc * xc, axis=-1, keepdims=True)
        hidden = xc / jnp.sqrt(var + 1e-5) * lng_ref[...] + lnb_ref[...]

        h3 = hidden.reshape(B, T, H)
        iota = jax.lax.broadcasted_iota(jnp.int32, (B, T), 1)
        validc = iota < NC

        def col_scores(mat, wcol_ref):
            wrep = jnp.broadcast_to(wcol_ref[...], (H, H))
            s = jnp.dot(mat, wrep, preferred_element_type=jnp.float32)
            return jnp.max(s.reshape(B, T, H), axis=-1)

        fwd_s = col_scores(hidden, fgw_ref)
        fwd_s = jnp.where(validc, fwd_s, NEG)

        def extract(scores, k):
            def body(_, carry):
                sc, m = carry
                mx = jnp.max(sc, axis=1, keepdims=True)
                eq = sc == mx
                idx = jnp.min(jnp.where(eq, iota, T), axis=1, keepdims=True)
                sel = iota == idx
                return jnp.where(sel, NEG, sc), jnp.maximum(
                    m, jnp.where(sel, 1.0, 0.0))
            _, mask = jax.lax.fori_loop(
                0, k, body, (scores, jnp.zeros((B, T), jnp.float32)))
            return mask > 0.5

        fwd_mask = extract(fwd_s, FWD)

        context = jnp.mean(h3, axis=1)
        g1lin = jnp.dot(hidden, rw1_ref[0:H, :],
                        preferred_element_type=jnp.float32).reshape(B, T, H)
        cb = jnp.dot(context, rw1_ref[H:2 * H, :],
                     preferred_element_type=jnp.float32) + rb1_ref[...]
        g1 = jnp.maximum(g1lin + cb.reshape(B, 1, H), 0.0)
        z = col_scores(g1.reshape(B * T, H), rw2_ref)
        z = jnp.where(
            jnp.logical_and(validc, jnp.logical_not(fwd_mask)), z, NEG)
        retro_mask = extract(z, RETRO)

        sel = jnp.logical_or(fwd_mask, retro_mask)

        q = jnp.dot(h3[:, T - 2, :], qw_ref[...],
                    preferred_element_type=jnp.float32) + qb_ref[...]
        h3r = h3.astype(jnp.bfloat16).astype(jnp.float32)
        qr = q.astype(jnp.bfloat16).astype(jnp.float32)
        att = jnp.sum(h3r * qr.reshape(B, 1, H), axis=-1)
        att = jnp.where(sel, att, NEG)
        mx = jnp.max(att, axis=1, keepdims=True)
        e = jnp.exp(att - mx)
        attn = e / jnp.sum(e, axis=1, keepdims=True)
        ctx_ref[...] = jnp.sum(h3 * attn.reshape(B, T, 1), axis=1)

    o_ref[...] = jnp.dot(ctx_ref[...], w_ref[...],
                         preferred_element_type=jnp.float32) + b_ref[...]


def _fused(h0, ffw1, ffb1, ffw2, ffb2, lng, lnb, fgw, rw1, rb1,
           rw2, qw, qb, out_w, out_b2):
    vocab = out_w.shape[1]
    whole = pl.BlockSpec(index_map=lambda i: (0, 0))
    return pl.pallas_call(
        _fused_body,
        grid=(pl.cdiv(vocab, _VTILE),),
        in_specs=[
            pl.BlockSpec((B * T, H), lambda i: (0, 0)),
            whole, whole, whole, whole, whole, whole, whole, whole,
            whole, whole, whole, whole,
            pl.BlockSpec((H, _VTILE), lambda i: (0, i)),
            pl.BlockSpec((1, _VTILE), lambda i: (0, i)),
        ],
        out_specs=pl.BlockSpec((B, _VTILE), lambda i: (0, i)),
        out_shape=jax.ShapeDtypeStruct((B, vocab), jnp.float32),
        scratch_shapes=[pltpu.VMEM((B, H), jnp.float32)],
        compiler_params=pltpu.CompilerParams(
            dimension_semantics=("arbitrary",)),
    )(h0, ffw1, ffb1, ffw2, ffb2, lng, lnb, fgw, rw1, rb1,
      rw2, qw, qb, out_w, out_b2)


def kernel(seq, emb, ff_w1, ff_b1, ff_w2, ff_b2, ln_g, ln_b, fg_w, fg_b,
           rev_w1, rev_b1, rev_w2, rev_b2, q_w, q_b, out_w, out_b):
    h0 = _sc_gather(emb, seq.reshape(1, B * T).astype(jnp.int32))
    return _fused(
        h0,
        ff_w1, ff_b1.reshape(1, 2 * H), ff_w2, ff_b2.reshape(1, H),
        ln_g.reshape(1, H), ln_b.reshape(1, H),
        fg_w,
        rev_w1, rev_b1.reshape(1, H),
        rev_w2,
        q_w, q_b.reshape(1, H),
        out_w, out_b.reshape(1, out_w.shape[1]),
    )


# speedup vs baseline: 1.3368x; 1.0572x over previous
import jax
import jax.numpy as jnp
from jax.experimental import pallas as pl
from jax.experimental.pallas import tpu as pltpu
from jax.experimental.pallas import tpu_sc as plsc

B = 16
T = 512
H = 128
FWD = 48
RETRO = 16
NC = T - 3
NEG = float("-inf")

_GATHER_WINDOW = 256


def _sc_gather(emb, seq_flat):
    n = seq_flat.shape[1]
    mesh = plsc.VectorSubcoreMesh(core_axis_name="core",
                                  subcore_axis_name="subcore")

    @pl.kernel(out_type=jax.ShapeDtypeStruct((n, emb.shape[1]), emb.dtype),
               mesh=mesh)
    def gather_kernel(x_hbm, i_hbm, o_hbm):
        def body(i_vmem, o_vmem):
            pltpu.sync_copy(x_hbm.at[i_vmem.at[0]], o_vmem)

        pltpu.emit_pipeline(
            body,
            grid=(n // _GATHER_WINDOW,),
            in_specs=[pl.BlockSpec((1, _GATHER_WINDOW),
                                   index_map=lambda i: (0, i))],
            out_specs=[pl.BlockSpec((_GATHER_WINDOW, emb.shape[1]),
                                    index_map=lambda i: (i, 0))],
            core_axis_name=("core", "subcore"),
            dimension_semantics=(pltpu.PARALLEL,),
        )(i_hbm, o_hbm)

    return gather_kernel(emb, seq_flat)


_VTILE = 8192


def _fused_body(h0_ref, ffw1_ref, ffb1_ref, ffw2_ref, ffb2_ref,
                lng_ref, lnb_ref, fgw_ref, rw1_ref,
                rb1_ref, rw2_ref, qw_ref, qb_ref,
                w_ref, b_ref, o_ref, ctx_ref):
    @pl.when(pl.program_id(0) == 0)
    def _compute_ctx():
        h0 = h0_ref[...]
        ff1 = jnp.maximum(
            jnp.dot(h0, ffw1_ref[...], preferred_element_type=jnp.float32)
            + ffb1_ref[...], 0.0)
        ff = jnp.dot(ff1, ffw2_ref[...],
                     preferred_element_type=jnp.float32) + ffb2_ref[...]
        x = h0 + ff
        mu = jnp.mean(x, axis=-1, keepdims=True)
        xc = x - mu
        var = jnp.mean(xc * xc, axis=-1, keepdims=True)
        hidden = xc / jnp.sqrt(var + 1e-5) * lng_ref[...] + lnb_ref[...]

        h3 = hidden.reshape(B, T, H)
        iota = jax.lax.broadcasted_iota(jnp.int32, (B, T), 1)
        validc = iota < NC

        def col_scores(mat, wcol_ref):
            wrep = jnp.broadcast_to(wcol_ref[...], (H, H))
            s = jnp.dot(mat, wrep, preferred_element_type=jnp.float32)
            return jnp.max(s.reshape(B, T, H), axis=-1)

        fwd_s = col_scores(hidden, fgw_ref)
        fwd_s = jnp.where(validc, fwd_s, NEG)

        def extract(scores, k):
            def body(_, carry):
                sc, m = carry
                mx = jnp.max(sc, axis=1, keepdims=True)
                eq = sc == mx
                idx = jnp.min(jnp.where(eq, iota, T), axis=1, keepdims=True)
                sel = iota == idx
                return jnp.where(sel, NEG, sc), jnp.maximum(
                    m, jnp.where(sel, 1.0, 0.0))
            _, mask = jax.lax.fori_loop(
                0, k, body, (scores, jnp.zeros((B, T), jnp.float32)))
            return mask > 0.5

        fwd_mask = extract(fwd_s, FWD)

        context = jnp.mean(h3, axis=1)
        g1lin = jnp.dot(hidden, rw1_ref[0:H, :],
                        preferred_element_type=jnp.float32).reshape(B, T, H)
        cb = jnp.dot(context, rw1_ref[H:2 * H, :],
                     preferred_element_type=jnp.float32) + rb1_ref[...]
        g1 = jnp.maximum(g1lin + cb.reshape(B, 1, H), 0.0)
        z = col_scores(g1.reshape(B * T, H), rw2_ref)
        z = jnp.where(
            jnp.logical_and(validc, jnp.logical_not(fwd_mask)), z, NEG)
        retro_mask = extract(z, RETRO)

        sel = jnp.logical_or(fwd_mask, retro_mask)

        q = jnp.dot(h3[:, T - 2, :], qw_ref[...],
                    preferred_element_type=jnp.float32) + qb_ref[...]
        h3r = h3.astype(jnp.bfloat16).astype(jnp.float32)
        qr = q.astype(jnp.bfloat16).astype(jnp.float32)
        att = jnp.sum(h3r * qr.reshape(B, 1, H), axis=-1)
        att = jnp.where(sel, att, NEG)
        mx = jnp.max(att, axis=1, keepdims=True)
        e = jnp.exp(att - mx)
        attn = e / jnp.sum(e, axis=1, keepdims=True)
        ctx_ref[...] = jnp.sum(h3 * attn.reshape(B, T, 1), axis=1)

    o_ref[...] = jnp.dot(ctx_ref[...], w_ref[...],
                         preferred_element_type=jnp.float32) + b_ref[...]


def _fused(h0, ffw1, ffb1, ffw2, ffb2, lng, lnb, fgw, rw1, rb1,
           rw2, qw, qb, out_w, out_b2):
    vocab = out_w.shape[1]
    whole = pl.BlockSpec(index_map=lambda i: (0, 0))
    return pl.pallas_call(
        _fused_body,
        grid=(pl.cdiv(vocab, _VTILE),),
        in_specs=[
            pl.BlockSpec((B * T, H), lambda i: (0, 0)),
            whole, whole, whole, whole, whole, whole, whole, whole,
            whole, whole, whole, whole,
            pl.BlockSpec((H, _VTILE), lambda i: (0, i)),
            pl.BlockSpec((1, _VTILE), lambda i: (0, i)),
        ],
        out_specs=pl.BlockSpec((B, _VTILE), lambda i: (0, i)),
        out_shape=jax.ShapeDtypeStruct((B, vocab), jnp.float32),
        scratch_shapes=[pltpu.VMEM((B, H), jnp.float32)],
        compiler_params=pltpu.CompilerParams(
            dimension_semantics=("arbitrary",)),
    )(h0, ffw1, ffb1, ffw2, ffb2, lng, lnb, fgw, rw1, rb1,
      rw2, qw, qb, out_w, out_b2)


def kernel(seq, emb, ff_w1, ff_b1, ff_w2, ff_b2, ln_g, ln_b, fg_w, fg_b,
           rev_w1, rev_b1, rev_w2, rev_b2, q_w, q_b, out_w, out_b):
    h0 = _sc_gather(emb, seq.reshape(1, B * T).astype(jnp.int32))
    return _fused(
        h0,
        ff_w1, ff_b1.reshape(1, 2 * H), ff_w2, ff_b2.reshape(1, H),
        ln_g.reshape(1, H), ln_b.reshape(1, H),
        fg_w,
        rev_w1, rev_b1.reshape(1, H),
        rev_w2,
        q_w, q_b.reshape(1, H),
        out_w, out_b.reshape(1, out_w.shape[1]),
    )


# speedup vs baseline: 1.3569x; 1.0151x over previous
import jax
import jax.numpy as jnp
from jax.experimental import pallas as pl
from jax.experimental.pallas import tpu as pltpu
from jax.experimental.pallas import tpu_sc as plsc

B = 16
T = 512
H = 128
FWD = 48
RETRO = 16
NC = T - 3
NEG = float("-inf")

_GATHER_WINDOW = 256


def _sc_gather(emb, seq_flat):
    n = seq_flat.shape[1]
    mesh = plsc.VectorSubcoreMesh(core_axis_name="core",
                                  subcore_axis_name="subcore")

    @pl.kernel(out_type=jax.ShapeDtypeStruct((n, emb.shape[1]), emb.dtype),
               mesh=mesh)
    def gather_kernel(x_hbm, i_hbm, o_hbm):
        def body(i_vmem, o_vmem):
            pltpu.sync_copy(x_hbm.at[i_vmem.at[0]], o_vmem)

        pltpu.emit_pipeline(
            body,
            grid=(n // _GATHER_WINDOW,),
            in_specs=[pl.BlockSpec((1, _GATHER_WINDOW),
                                   index_map=lambda i: (0, i))],
            out_specs=[pl.BlockSpec((_GATHER_WINDOW, emb.shape[1]),
                                    index_map=lambda i: (i, 0))],
            core_axis_name=("core", "subcore"),
            dimension_semantics=(pltpu.PARALLEL,),
        )(i_hbm, o_hbm)

    return gather_kernel(emb, seq_flat)


_VTILE = 16384


def _fused_body(h0_ref, ffw1_ref, ffb1_ref, ffw2_ref, ffb2_ref,
                lng_ref, lnb_ref, fgw_ref, rw1_ref,
                rb1_ref, rw2_ref, qw_ref, qb_ref,
                w_ref, b_ref, o_ref, ctx_ref):
    @pl.when(pl.program_id(0) == 0)
    def _compute_ctx():
        h0 = h0_ref[...]
        ff1 = jnp.maximum(
            jnp.dot(h0, ffw1_ref[...], preferred_element_type=jnp.float32)
            + ffb1_ref[...], 0.0)
        ff = jnp.dot(ff1, ffw2_ref[...],
                     preferred_element_type=jnp.float32) + ffb2_ref[...]
        x = h0 + ff
        mu = jnp.mean(x, axis=-1, keepdims=True)
        xc = x - mu
        var = jnp.mean(xc * xc, axis=-1, keepdims=True)
        hidden = xc / jnp.sqrt(var + 1e-5) * lng_ref[...] + lnb_ref[...]

        h3 = hidden.reshape(B, T, H)
        iota = jax.lax.broadcasted_iota(jnp.int32, (B, T), 1)
        validc = iota < NC

        def col_scores(mat, wcol_ref):
            wrep = jnp.broadcast_to(wcol_ref[...], (H, H))
            s = jnp.dot(mat, wrep, preferred_element_type=jnp.float32)
            return jnp.max(s.reshape(B, T, H), axis=-1)

        fwd_s = col_scores(hidden, fgw_ref)
        fwd_s = jnp.where(validc, fwd_s, NEG)

        def extract(scores, k):
            def body(_, carry):
                sc, m = carry
                mx = jnp.max(sc, axis=1, keepdims=True)
                eq = sc == mx
                idx = jnp.min(jnp.where(eq, iota, T), axis=1, keepdims=True)
                sel = iota == idx
                return jnp.where(sel, NEG, sc), jnp.maximum(
                    m, jnp.where(sel, 1.0, 0.0))
            _, mask = jax.lax.fori_loop(
                0, k, body, (scores, jnp.zeros((B, T), jnp.float32)))
            return mask > 0.5

        fwd_mask = extract(fwd_s, FWD)

        context = jnp.mean(h3, axis=1)
        g1lin = jnp.dot(hidden, rw1_ref[0:H, :],
                        preferred_element_type=jnp.float32).reshape(B, T, H)
        cb = jnp.dot(context, rw1_ref[H:2 * H, :],
                     preferred_element_type=jnp.float32) + rb1_ref[...]
        g1 = jnp.maximum(g1lin + cb.reshape(B, 1, H), 0.0)
        z = col_scores(g1.reshape(B * T, H), rw2_ref)
        z = jnp.where(
            jnp.logical_and(validc, jnp.logical_not(fwd_mask)), z, NEG)
        retro_mask = extract(z, RETRO)

        sel = jnp.logical_or(fwd_mask, retro_mask)

        q = jnp.dot(h3[:, T - 2, :], qw_ref[...],
                    preferred_element_type=jnp.float32) + qb_ref[...]
        h3r = h3.astype(jnp.bfloat16).astype(jnp.float32)
        qr = q.astype(jnp.bfloat16).astype(jnp.float32)
        att = jnp.sum(h3r * qr.reshape(B, 1, H), axis=-1)
        att = jnp.where(sel, att, NEG)
        mx = jnp.max(att, axis=1, keepdims=True)
        e = jnp.exp(att - mx)
        attn = e / jnp.sum(e, axis=1, keepdims=True)
        ctx_ref[...] = jnp.sum(h3 * attn.reshape(B, T, 1), axis=1)

    o_ref[...] = jnp.dot(ctx_ref[...], w_ref[...],
                         preferred_element_type=jnp.float32) + b_ref[...]


def _fused(h0, ffw1, ffb1, ffw2, ffb2, lng, lnb, fgw, rw1, rb1,
           rw2, qw, qb, out_w, out_b2):
    vocab = out_w.shape[1]
    whole = pl.BlockSpec(index_map=lambda i: (0, 0))
    return pl.pallas_call(
        _fused_body,
        grid=(pl.cdiv(vocab, _VTILE),),
        in_specs=[
            pl.BlockSpec((B * T, H), lambda i: (0, 0)),
            whole, whole, whole, whole, whole, whole, whole, whole,
            whole, whole, whole, whole,
            pl.BlockSpec((H, _VTILE), lambda i: (0, i)),
            pl.BlockSpec((1, _VTILE), lambda i: (0, i)),
        ],
        out_specs=pl.BlockSpec((B, _VTILE), lambda i: (0, i)),
        out_shape=jax.ShapeDtypeStruct((B, vocab), jnp.float32),
        scratch_shapes=[pltpu.VMEM((B, H), jnp.float32)],
        compiler_params=pltpu.CompilerParams(
            dimension_semantics=("arbitrary",)),
    )(h0, ffw1, ffb1, ffw2, ffb2, lng, lnb, fgw, rw1, rb1,
      rw2, qw, qb, out_w, out_b2)


def kernel(seq, emb, ff_w1, ff_b1, ff_w2, ff_b2, ln_g, ln_b, fg_w, fg_b,
           rev_w1, rev_b1, rev_w2, rev_b2, q_w, q_b, out_w, out_b):
    h0 = _sc_gather(emb, seq.reshape(1, B * T).astype(jnp.int32))
    return _fused(
        h0,
        ff_w1, ff_b1.reshape(1, 2 * H), ff_w2, ff_b2.reshape(1, H),
        ln_g.reshape(1, H), ln_b.reshape(1, H),
        fg_w,
        rev_w1, rev_b1.reshape(1, H),
        rev_w2,
        q_w, q_b.reshape(1, H),
        out_w, out_b.reshape(1, out_w.shape[1]),
    )


# speedup vs baseline: 1.3597x; 1.0020x over previous
import jax
import jax.numpy as jnp
from jax.experimental import pallas as pl
from jax.experimental.pallas import tpu as pltpu
from jax.experimental.pallas import tpu_sc as plsc

B = 16
T = 512
H = 128
FWD = 48
RETRO = 16
NC = T - 3
NEG = float("-inf")

_GATHER_WINDOW = 256


def _sc_gather(emb, seq_flat):
    n = seq_flat.shape[1]
    mesh = plsc.VectorSubcoreMesh(core_axis_name="core",
                                  subcore_axis_name="subcore")

    @pl.kernel(out_type=jax.ShapeDtypeStruct((n, emb.shape[1]), emb.dtype),
               mesh=mesh)
    def gather_kernel(x_hbm, i_hbm, o_hbm):
        def body(i_vmem, o_vmem):
            pltpu.sync_copy(x_hbm.at[i_vmem.at[0]], o_vmem)

        pltpu.emit_pipeline(
            body,
            grid=(n // _GATHER_WINDOW,),
            in_specs=[pl.BlockSpec((1, _GATHER_WINDOW),
                                   index_map=lambda i: (0, i))],
            out_specs=[pl.BlockSpec((_GATHER_WINDOW, emb.shape[1]),
                                    index_map=lambda i: (i, 0))],
            core_axis_name=("core", "subcore"),
            dimension_semantics=(pltpu.PARALLEL,),
        )(i_hbm, o_hbm)

    return gather_kernel(emb, seq_flat)


_VTILE = 20480


def _fused_body(h0_ref, ffw1_ref, ffb1_ref, ffw2_ref, ffb2_ref,
                lng_ref, lnb_ref, fgw_ref, rw1_ref,
                rb1_ref, rw2_ref, qw_ref, qb_ref,
                w_ref, b_ref, o_ref, ctx_ref):
    @pl.when(pl.program_id(0) == 0)
    def _compute_ctx():
        h0 = h0_ref[...]
        ff1 = jnp.maximum(
            jnp.dot(h0, ffw1_ref[...], preferred_element_type=jnp.float32)
            + ffb1_ref[...], 0.0)
        ff = jnp.dot(ff1, ffw2_ref[...],
                     preferred_element_type=jnp.float32) + ffb2_ref[...]
        x = h0 + ff
        mu = jnp.mean(x, axis=-1, keepdims=True)
        xc = x - mu
        var = jnp.mean(xc * xc, axis=-1, keepdims=True)
        hidden = xc / jnp.sqrt(var + 1e-5) * lng_ref[...] + lnb_ref[...]

        h3 = hidden.reshape(B, T, H)
        iota = jax.lax.broadcasted_iota(jnp.int32, (B, T), 1)
        validc = iota < NC

        def col_scores(mat, wcol_ref):
            wrep = jnp.broadcast_to(wcol_ref[...], (H, H))
            s = jnp.dot(mat, wrep, preferred_element_type=jnp.float32)
            return jnp.max(s.reshape(B, T, H), axis=-1)

        fwd_s = col_scores(hidden, fgw_ref)
        fwd_s = jnp.where(validc, fwd_s, NEG)

        def extract(scores, k):
            def body(_, carry):
                sc, m = carry
                mx = jnp.max(sc, axis=1, keepdims=True)
                eq = sc == mx
                idx = jnp.min(jnp.where(eq, iota, T), axis=1, keepdims=True)
                sel = iota == idx
                return jnp.where(sel, NEG, sc), jnp.maximum(
                    m, jnp.where(sel, 1.0, 0.0))
            _, mask = jax.lax.fori_loop(
                0, k, body, (scores, jnp.zeros((B, T), jnp.float32)))
            return mask > 0.5

        fwd_mask = extract(fwd_s, FWD)

        context = jnp.mean(h3, axis=1)
        g1lin = jnp.dot(hidden, rw1_ref[0:H, :],
                        preferred_element_type=jnp.float32).reshape(B, T, H)
        cb = jnp.dot(context, rw1_ref[H:2 * H, :],
                     preferred_element_type=jnp.float32) + rb1_ref[...]
        g1 = jnp.maximum(g1lin + cb.reshape(B, 1, H), 0.0)
        z = col_scores(g1.reshape(B * T, H), rw2_ref)
        z = jnp.where(
            jnp.logical_and(validc, jnp.logical_not(fwd_mask)), z, NEG)
        retro_mask = extract(z, RETRO)

        sel = jnp.logical_or(fwd_mask, retro_mask)

        q = jnp.dot(h3[:, T - 2, :], qw_ref[...],
                    preferred_element_type=jnp.float32) + qb_ref[...]
        h3r = h3.astype(jnp.bfloat16).astype(jnp.float32)
        qr = q.astype(jnp.bfloat16).astype(jnp.float32)
        att = jnp.sum(h3r * qr.reshape(B, 1, H), axis=-1)
        att = jnp.where(sel, att, NEG)
        mx = jnp.max(att, axis=1, keepdims=True)
        e = jnp.exp(att - mx)
        attn = e / jnp.sum(e, axis=1, keepdims=True)
        ctx_ref[...] = jnp.sum(h3 * attn.reshape(B, T, 1), axis=1)

    o_ref[...] = jnp.dot(ctx_ref[...], w_ref[...],
                         preferred_element_type=jnp.float32) + b_ref[...]


def _fused(h0, ffw1, ffb1, ffw2, ffb2, lng, lnb, fgw, rw1, rb1,
           rw2, qw, qb, out_w, out_b2):
    vocab = out_w.shape[1]
    whole = pl.BlockSpec(index_map=lambda i: (0, 0))
    return pl.pallas_call(
        _fused_body,
        grid=(pl.cdiv(vocab, _VTILE),),
        in_specs=[
            pl.BlockSpec((B * T, H), lambda i: (0, 0)),
            whole, whole, whole, whole, whole, whole, whole, whole,
            whole, whole, whole, whole,
            pl.BlockSpec((H, _VTILE), lambda i: (0, i)),
            pl.BlockSpec((1, _VTILE), lambda i: (0, i)),
        ],
        out_specs=pl.BlockSpec((B, _VTILE), lambda i: (0, i)),
        out_shape=jax.ShapeDtypeStruct((B, vocab), jnp.float32),
        scratch_shapes=[pltpu.VMEM((B, H), jnp.float32)],
        compiler_params=pltpu.CompilerParams(
            dimension_semantics=("arbitrary",)),
    )(h0, ffw1, ffb1, ffw2, ffb2, lng, lnb, fgw, rw1, rb1,
      rw2, qw, qb, out_w, out_b2)


def kernel(seq, emb, ff_w1, ff_b1, ff_w2, ff_b2, ln_g, ln_b, fg_w, fg_b,
           rev_w1, rev_b1, rev_w2, rev_b2, q_w, q_b, out_w, out_b):
    h0 = _sc_gather(emb, seq.reshape(1, B * T).astype(jnp.int32))
    return _fused(
        h0,
        ff_w1, ff_b1.reshape(1, 2 * H), ff_w2, ff_b2.reshape(1, H),
        ln_g.reshape(1, H), ln_b.reshape(1, H),
        fg_w,
        rev_w1, rev_b1.reshape(1, H),
        rev_w2,
        q_w, q_b.reshape(1, H),
        out_w, out_b.reshape(1, out_w.shape[1]),
    )
